# speedup vs baseline: 2.7786x; 2.7786x over previous
import functools

import jax
import jax.numpy as jnp
from jax import lax
from jax.experimental import pallas as pl
from jax.experimental.pallas import tpu as pltpu
from jax.experimental.pallas import tpu_sc as plsc

NUM_EMBEDDINGS = 256
EMBEDDING_DIM = 256

_info = plsc.get_sparse_core_info()
_NC, _NS = _info.num_cores, _info.num_subcores
_NW = _NC * _NS

_B = 1024 * 200
_BPW = _B // _NW
_CHUNK = 128
_STEPS = _BPW // _CHUNK


def _make_kernel():
  mesh = plsc.VectorSubcoreMesh(core_axis_name="c", subcore_axis_name="s")

  @functools.partial(
      pl.kernel,
      mesh=mesh,
      out_type=jax.ShapeDtypeStruct((_B, EMBEDDING_DIM), jnp.float32),
      scratch_types=[
          pltpu.VMEM((_BPW,), jnp.int32),
          pltpu.VMEM((_CHUNK, EMBEDDING_DIM), jnp.float32),
          pltpu.SemaphoreType.DMA,
      ],
  )
  def k(idx_hbm, table_hbm, out_hbm, idx_v, rows_v, sem):
    wid = lax.axis_index("s") * _NC + lax.axis_index("c")
    base = wid * _BPW
    pltpu.sync_copy(idx_hbm.at[pl.ds(base, _BPW)], idx_v)

    def body(g, _):
      off = g * _CHUNK
      pltpu.async_copy(
          table_hbm.at[idx_v.at[pl.ds(off, _CHUNK)]], rows_v, sem
      ).wait()
      pltpu.sync_copy(rows_v, out_hbm.at[pl.ds(base + off, _CHUNK)])
      return 0

    lax.fori_loop(0, _STEPS, body, 0)

  return k


_kernel = _make_kernel()


@jax.jit
def kernel(text_inputs, embedding_table):
  idx = text_inputs.reshape(-1).astype(jnp.int32)
  out = _kernel(idx, embedding_table)
  return out.reshape(text_inputs.shape + (EMBEDDING_DIM,))
